# baseline (speedup 1.0000x reference)
import functools

import jax
import jax.numpy as jnp
from jax import lax
from jax.experimental import pallas as pl
from jax.experimental.pallas import tpu as pltpu
from jax.experimental.pallas import tpu_sc as plsc

_N = 10000
_NPAD = 10112
_E = 320000
_CHUNK = 128
_NS = 16
_CPT = 160
_EPAD = _CHUNK * _NS * _CPT
_NCHUNK = _EPAD // _CHUNK
_RPT = _NPAD // _NS
_ZROWS = 128
_BM = 400


def _leaky(t):
    return jnp.where(t >= 0, t, 0.01 * t)


_DCPW = _NCHUNK // 32


def _deg_body(dst_hbm, out_hbm, didx_all, d0, d1, d2, d3, d4, d5, d6, d7,
              ones, zbuf, acc, sem):
    dbufs = (d0, d1, d2, d3, d4, d5, d6, d7)
    c = lax.axis_index("c")
    s = lax.axis_index("s")
    wid = c * _NS + s

    def fillz(i, carry):
        zbuf[pl.ds(i * 16, 16)] = jnp.zeros((16,), jnp.float32)
        return carry

    lax.fori_loop(0, 40, fillz, 0)

    def fill1(i, carry):
        ones[pl.ds(i * 16, 16)] = jnp.ones((16,), jnp.float32)
        return carry

    lax.fori_loop(0, _CHUNK // 16, fill1, 0)

    pltpu.sync_copy(zbuf.at[pl.ds(0, _RPT)], acc.at[pl.ds(s * _RPT, _RPT)])
    pltpu.sync_copy(dst_hbm.at[pl.ds(wid * _DCPW, _DCPW)], didx_all)
    plsc.subcore_barrier()

    def wave(j8, carry):
        for b in range(8):
            j = j8 * 8 + b
            for t in range(_CHUNK // 16):
                dbufs[b][pl.ds(t * 16, 16)] = didx_all[j, pl.ds(t * 16, 16)]
            pltpu.async_copy(ones, acc.at[dbufs[b]], sem, add=True)
        for b in range(8):
            pltpu.make_async_copy(ones, acc.at[dbufs[0]], sem).wait()
        return carry

    lax.fori_loop(0, _DCPW // 8, wave, 0)
    plsc.subcore_barrier()

    @pl.when(s == 0)
    def _():
        pltpu.sync_copy(acc, out_hbm.at[c])


_deg = functools.partial(
    pl.kernel,
    out_type=jax.ShapeDtypeStruct((2, _NPAD), jnp.float32),
    mesh=plsc.VectorSubcoreMesh(core_axis_name="c", subcore_axis_name="s"),
    scratch_types=[
        pltpu.VMEM((_DCPW, _CHUNK), jnp.int32),
    ] + [pltpu.VMEM((_CHUNK,), jnp.int32)] * 8 + [
        pltpu.VMEM((_CHUNK,), jnp.float32),
        pltpu.VMEM((640,), jnp.float32),
        pltpu.VMEM_SHARED((_NPAD,), jnp.float32),
        pltpu.SemaphoreType.DMA,
    ],
)(_deg_body)


def _zero_acc(zbuf, acc, s, hw):
    def fillz(i, carry):
        for l in range(hw // 16):
            zbuf[i, pl.ds(l * 16, 16)] = jnp.zeros((16,), jnp.float32)
        return carry

    lax.fori_loop(0, _ZROWS, fillz, 0)
    for k in range(_RPT // _ZROWS):
        pltpu.sync_copy(zbuf, acc.at[pl.ds(s * _RPT + k * _ZROWS, _ZROWS)])
    tail = _RPT - (_RPT // _ZROWS) * _ZROWS
    if tail:
        pltpu.sync_copy(
            zbuf.at[pl.ds(0, tail)],
            acc.at[pl.ds(s * _RPT + (_RPT // _ZROWS) * _ZROWS, tail)])


def _scatter128_body(y_hbm, src_hbm, dst_hbm, out0, out1,
                     si0, si1, gi0, gi1, di0, di1,
                     rb0, rb1, zbuf, acc, gs0, gs1):
    c = lax.axis_index("c")
    s = lax.axis_index("s")

    _zero_acc(zbuf, acc, s, 128)
    plsc.subcore_barrier()

    def chunk(j, carry):
        ch = s + _NS * j

        @pl.when(ch < _NCHUNK)
        def _():
            pltpu.sync_copy(src_hbm.at[pl.ds(ch * _CHUNK, _CHUNK)], si0)
            pltpu.sync_copy(dst_hbm.at[pl.ds(ch * _CHUNK, _CHUNK)], di0)
            for t in range(_CHUNK // 16):
                sv = si0[pl.ds(t * 16, 16)]
                gi0[pl.ds(t * 16, 16)] = sv * 2 + c
            pltpu.async_copy(y_hbm.at[gi0], rb0, gs0).wait()
            pltpu.sync_copy(rb0, acc.at[di0], add=True)

        return carry

    lax.fori_loop(0, _CPT, chunk, 0)
    plsc.subcore_barrier()

    @pl.when(c == 0)
    def _():
        pltpu.sync_copy(acc.at[pl.ds(s * _RPT, _RPT)],
                        out0.at[pl.ds(s * _RPT, _RPT)])

    @pl.when(c == 1)
    def _():
        pltpu.sync_copy(acc.at[pl.ds(s * _RPT, _RPT)],
                        out1.at[pl.ds(s * _RPT, _RPT)])


_scatter128 = functools.partial(
    pl.kernel,
    out_type=(jax.ShapeDtypeStruct((_NPAD, 128), jnp.float32),
              jax.ShapeDtypeStruct((_NPAD, 128), jnp.float32)),
    mesh=plsc.VectorSubcoreMesh(core_axis_name="c", subcore_axis_name="s"),
    scratch_types=[
        pltpu.VMEM((_CHUNK,), jnp.int32),
        pltpu.VMEM((_CHUNK,), jnp.int32),
        pltpu.VMEM((_CHUNK,), jnp.int32),
        pltpu.VMEM((_CHUNK,), jnp.int32),
        pltpu.VMEM((_CHUNK,), jnp.int32),
        pltpu.VMEM((_CHUNK,), jnp.int32),
        pltpu.VMEM((_CHUNK, 128), jnp.float32),
        pltpu.VMEM((_CHUNK, 128), jnp.float32),
        pltpu.VMEM((_ZROWS, 128), jnp.float32),
        pltpu.VMEM_SHARED((_NPAD, 128), jnp.float32),
        pltpu.SemaphoreType.DMA,
        pltpu.SemaphoreType.DMA,
    ],
)(_scatter128_body)


_ECPT = _NCHUNK // 32


def _scatter_edges_body(y_hbm, src_hbm, dst_hbm, out0, out1,
                        si0, si1, di0, di1,
                        rb0, rb1, zbuf, acc, gs0, gs1):
    c = lax.axis_index("c")
    s = lax.axis_index("s")

    _zero_acc(zbuf, acc, s, 128)
    plsc.subcore_barrier()

    def chunk(j, carry):
        half = s + _NS * j

        @pl.when(half < _NCHUNK // 2)
        def _():
            ch = 2 * half + c
            pltpu.sync_copy(src_hbm.at[pl.ds(ch * _CHUNK, _CHUNK)], si0)
            pltpu.sync_copy(dst_hbm.at[pl.ds(ch * _CHUNK, _CHUNK)], di0)
            pltpu.async_copy(y_hbm.at[si0], rb0, gs0).wait()
            pltpu.sync_copy(rb0, acc.at[di0], add=True)

        return carry

    lax.fori_loop(0, _ECPT, chunk, 0)
    plsc.subcore_barrier()

    @pl.when(c == 0)
    def _():
        pltpu.sync_copy(acc.at[pl.ds(s * _RPT, _RPT)],
                        out0.at[pl.ds(s * _RPT, _RPT)])

    @pl.when(c == 1)
    def _():
        pltpu.sync_copy(acc.at[pl.ds(s * _RPT, _RPT)],
                        out1.at[pl.ds(s * _RPT, _RPT)])


_scatter_edges = functools.partial(
    pl.kernel,
    out_type=(jax.ShapeDtypeStruct((_NPAD, 128), jnp.float32),
              jax.ShapeDtypeStruct((_NPAD, 128), jnp.float32)),
    mesh=plsc.VectorSubcoreMesh(core_axis_name="c", subcore_axis_name="s"),
    scratch_types=[
        pltpu.VMEM((_CHUNK,), jnp.int32),
        pltpu.VMEM((_CHUNK,), jnp.int32),
        pltpu.VMEM((_CHUNK,), jnp.int32),
        pltpu.VMEM((_CHUNK,), jnp.int32),
        pltpu.VMEM((_CHUNK, 128), jnp.float32),
        pltpu.VMEM((_CHUNK, 128), jnp.float32),
        pltpu.VMEM((_ZROWS, 128), jnp.float32),
        pltpu.VMEM_SHARED((_NPAD, 128), jnp.float32),
        pltpu.SemaphoreType.DMA,
        pltpu.SemaphoreType.DMA,
    ],
)(_scatter_edges_body)


def _dense_body(x_ref, w1_ref, b1_ref, w2_ref, b2_ref, wc1_ref, deg_ref, y_ref):
    h = _leaky(jnp.dot(x_ref[...], w1_ref[...],
                       preferred_element_type=jnp.float32) + b1_ref[...])
    h = _leaky(jnp.dot(h, w2_ref[...],
                       preferred_element_type=jnp.float32) + b2_ref[...])
    xw = jnp.dot(h, wc1_ref[...], preferred_element_type=jnp.float32)
    dis = lax.rsqrt(deg_ref[...] + 1.0)
    y_ref[...] = xw * dis


def _mid_body(agg0_ref, agg1_ref, y_ref, deg_ref, bc1_ref, wc2_ref,
              e1_ref, y2_ref):
    dis = lax.rsqrt(deg_ref[...] + 1.0)
    agg = jnp.concatenate([agg0_ref[...], agg1_ref[...]], axis=1)
    e1 = _leaky(dis * (agg + y_ref[...]) + bc1_ref[...])
    e1_ref[...] = e1
    y2_ref[...] = jnp.dot(e1, wc2_ref[...],
                          preferred_element_type=jnp.float32) * dis


def _final_body(agg0_ref, agg1_ref, y2_ref, deg_ref, bc2_ref, e2_ref):
    dis = lax.rsqrt(deg_ref[...] + 1.0)
    agg = agg0_ref[...] + agg1_ref[...]
    e2_ref[...] = _leaky(dis * (agg + y2_ref[...]) + bc2_ref[...])


def _row_spec(w):
    return pl.BlockSpec((_BM, w), lambda i: (i, 0))


def _full_spec(h, w):
    return pl.BlockSpec((h, w), lambda i: (0, 0))


_GRID = _N // _BM

_dense = pl.pallas_call(
    _dense_body,
    grid=(_GRID,),
    in_specs=[_row_spec(128), _full_spec(128, 256), _full_spec(1, 256),
              _full_spec(256, 256), _full_spec(1, 256), _full_spec(256, 256),
              _row_spec(1)],
    out_specs=_row_spec(256),
    out_shape=jax.ShapeDtypeStruct((_N, 256), jnp.float32),
)

_mid = pl.pallas_call(
    _mid_body,
    grid=(_GRID,),
    in_specs=[_row_spec(128), _row_spec(128), _row_spec(256), _row_spec(1),
              _full_spec(1, 256), _full_spec(256, 128)],
    out_specs=(_row_spec(256), _row_spec(128)),
    out_shape=(jax.ShapeDtypeStruct((_N, 256), jnp.float32),
               jax.ShapeDtypeStruct((_N, 128), jnp.float32)),
)

_final = pl.pallas_call(
    _final_body,
    grid=(_GRID,),
    in_specs=[_row_spec(128), _row_spec(128), _row_spec(128), _row_spec(1),
              _full_spec(1, 128)],
    out_specs=_row_spec(128),
    out_shape=jax.ShapeDtypeStruct((_N, 128), jnp.float32),
)


def kernel(x, edge_index, W1, b1, W2, b2, Wc1, bc1, Wc2, bc2, prev1, prev2):
    npad = _EPAD - _E
    src = jnp.concatenate([edge_index[0], jnp.zeros((npad,), jnp.int32)])
    dst = jnp.concatenate([edge_index[1], jnp.full((npad,), _N, jnp.int32)])
    dst2 = dst.reshape(_NCHUNK, _CHUNK)
    dp = _deg(dst2)
    deg_col = (dp[0, :_N] + dp[1, :_N]).reshape(_N, 1)
    y1 = _dense(x, W1, b1.reshape(1, -1), W2, b2.reshape(1, -1), Wc1, deg_col)
    a0, a1 = _scatter128(y1.reshape(2 * _N, 128), src, dst)
    e1, y2 = _mid(a0[:_N], a1[:_N], y1, deg_col, bc1.reshape(1, -1), Wc2)
    c0, c1 = _scatter_edges(y2, src, dst)
    e2 = _final(c0[:_N], c1[:_N], y2, deg_col, bc2.reshape(1, -1))
    return (e1, e2)


# speedup vs baseline: 1.0055x; 1.0055x over previous
import functools

import jax
import jax.numpy as jnp
from jax import lax
from jax.experimental import pallas as pl
from jax.experimental.pallas import tpu as pltpu
from jax.experimental.pallas import tpu_sc as plsc

_N = 10000
_NPAD = 10112
_E = 320000
_CHUNK = 128
_NS = 16
_CPT = 160
_EPAD = _CHUNK * _NS * _CPT
_NCHUNK = _EPAD // _CHUNK
_RPT = _NPAD // _NS
_ZROWS = 128
_BM = 400


def _leaky(t):
    return jnp.where(t >= 0, t, 0.01 * t)


_DCPW = _NCHUNK // 32


def _deg_body(dst_hbm, out_hbm, didx_all, d0, d1, d2, d3, d4, d5, d6, d7,
              ones, zbuf, acc, sem):
    dbufs = (d0, d1, d2, d3, d4, d5, d6, d7)
    c = lax.axis_index("c")
    s = lax.axis_index("s")
    wid = c * _NS + s

    def fillz(i, carry):
        zbuf[pl.ds(i * 16, 16)] = jnp.zeros((16,), jnp.float32)
        return carry

    lax.fori_loop(0, 40, fillz, 0)

    def fill1(i, carry):
        ones[pl.ds(i * 16, 16)] = jnp.ones((16,), jnp.float32)
        return carry

    lax.fori_loop(0, _CHUNK // 16, fill1, 0)

    pltpu.sync_copy(zbuf.at[pl.ds(0, _RPT)], acc.at[pl.ds(s * _RPT, _RPT)])
    pltpu.sync_copy(dst_hbm.at[pl.ds(wid * _DCPW, _DCPW)], didx_all)
    plsc.subcore_barrier()

    def wave(j8, carry):
        for b in range(8):
            j = j8 * 8 + b
            for t in range(_CHUNK // 16):
                dbufs[b][pl.ds(t * 16, 16)] = didx_all[j, pl.ds(t * 16, 16)]
            pltpu.async_copy(ones, acc.at[dbufs[b]], sem, add=True)
        for b in range(8):
            pltpu.make_async_copy(ones, acc.at[dbufs[0]], sem).wait()
        return carry

    lax.fori_loop(0, _DCPW // 8, wave, 0)
    plsc.subcore_barrier()

    @pl.when(s == 0)
    def _():
        pltpu.sync_copy(acc, out_hbm.at[c])


_deg = functools.partial(
    pl.kernel,
    out_type=jax.ShapeDtypeStruct((2, _NPAD), jnp.float32),
    mesh=plsc.VectorSubcoreMesh(core_axis_name="c", subcore_axis_name="s"),
    scratch_types=[
        pltpu.VMEM((_DCPW, _CHUNK), jnp.int32),
    ] + [pltpu.VMEM((_CHUNK,), jnp.int32)] * 8 + [
        pltpu.VMEM((_CHUNK,), jnp.float32),
        pltpu.VMEM((640,), jnp.float32),
        pltpu.VMEM_SHARED((_NPAD,), jnp.float32),
        pltpu.SemaphoreType.DMA,
    ],
)(_deg_body)


def _zero_acc(zbuf, acc, s, hw):
    def fillz(i, carry):
        for l in range(hw // 16):
            zbuf[i, pl.ds(l * 16, 16)] = jnp.zeros((16,), jnp.float32)
        return carry

    lax.fori_loop(0, _ZROWS, fillz, 0)
    for k in range(_RPT // _ZROWS):
        pltpu.sync_copy(zbuf, acc.at[pl.ds(s * _RPT + k * _ZROWS, _ZROWS)])
    tail = _RPT - (_RPT // _ZROWS) * _ZROWS
    if tail:
        pltpu.sync_copy(
            zbuf.at[pl.ds(0, tail)],
            acc.at[pl.ds(s * _RPT + (_RPT // _ZROWS) * _ZROWS, tail)])


def _scatter128_body(y_hbm, src_hbm, dst_hbm, out0, out1,
                     si0, si1, gi0, gi1, di0, di1,
                     rb0, rb1, zbuf, acc, gs0, gs1):
    c = lax.axis_index("c")
    s = lax.axis_index("s")

    _zero_acc(zbuf, acc, s, 128)
    plsc.subcore_barrier()

    def chunk(j, carry):
        ch = s + _NS * j

        @pl.when(ch < _NCHUNK)
        def _():
            pltpu.sync_copy(src_hbm.at[pl.ds(ch * _CHUNK, _CHUNK)], si0)
            pltpu.sync_copy(dst_hbm.at[pl.ds(ch * _CHUNK, _CHUNK)], di0)
            for t in range(_CHUNK // 16):
                sv = si0[pl.ds(t * 16, 16)]
                gi0[pl.ds(t * 16, 16)] = sv * 2 + c
            pltpu.async_copy(y_hbm.at[gi0], rb0, gs0).wait()
            pltpu.sync_copy(rb0, acc.at[di0], add=True)

        return carry

    lax.fori_loop(0, _CPT, chunk, 0)
    plsc.subcore_barrier()

    @pl.when(c == 0)
    def _():
        pltpu.sync_copy(acc.at[pl.ds(s * _RPT, _RPT)],
                        out0.at[pl.ds(s * _RPT, _RPT)])

    @pl.when(c == 1)
    def _():
        pltpu.sync_copy(acc.at[pl.ds(s * _RPT, _RPT)],
                        out1.at[pl.ds(s * _RPT, _RPT)])


_scatter128 = functools.partial(
    pl.kernel,
    out_type=(jax.ShapeDtypeStruct((_NPAD, 128), jnp.float32),
              jax.ShapeDtypeStruct((_NPAD, 128), jnp.float32)),
    mesh=plsc.VectorSubcoreMesh(core_axis_name="c", subcore_axis_name="s"),
    scratch_types=[
        pltpu.VMEM((_CHUNK,), jnp.int32),
        pltpu.VMEM((_CHUNK,), jnp.int32),
        pltpu.VMEM((_CHUNK,), jnp.int32),
        pltpu.VMEM((_CHUNK,), jnp.int32),
        pltpu.VMEM((_CHUNK,), jnp.int32),
        pltpu.VMEM((_CHUNK,), jnp.int32),
        pltpu.VMEM((_CHUNK, 128), jnp.float32),
        pltpu.VMEM((_CHUNK, 128), jnp.float32),
        pltpu.VMEM((_ZROWS, 128), jnp.float32),
        pltpu.VMEM_SHARED((_NPAD, 128), jnp.float32),
        pltpu.SemaphoreType.DMA,
        pltpu.SemaphoreType.DMA,
    ],
)(_scatter128_body)


_ECPT = _NCHUNK // 32


def _scatter_edges_body(y_hbm, src_hbm, dst_hbm, out0, out1,
                        si0, si1, di0, di1,
                        rb0, rb1, zbuf, acc, gs0, gs1):
    c = lax.axis_index("c")
    s = lax.axis_index("s")

    _zero_acc(zbuf, acc, s, 128)
    plsc.subcore_barrier()

    def chunk(j, carry):
        half = s + _NS * j

        @pl.when(half < _NCHUNK // 2)
        def _():
            ch = 2 * half + c
            pltpu.sync_copy(src_hbm.at[pl.ds(ch * _CHUNK, _CHUNK)], si0)
            pltpu.sync_copy(dst_hbm.at[pl.ds(ch * _CHUNK, _CHUNK)], di0)
            pltpu.async_copy(y_hbm.at[si0], rb0, gs0).wait()
            pltpu.sync_copy(rb0, acc.at[di0], add=True)

        return carry

    lax.fori_loop(0, _ECPT, chunk, 0)
    plsc.subcore_barrier()

    @pl.when(c == 0)
    def _():
        pltpu.sync_copy(acc.at[pl.ds(s * _RPT, _RPT)],
                        out0.at[pl.ds(s * _RPT, _RPT)])

    @pl.when(c == 1)
    def _():
        pltpu.sync_copy(acc.at[pl.ds(s * _RPT, _RPT)],
                        out1.at[pl.ds(s * _RPT, _RPT)])


_scatter_edges = functools.partial(
    pl.kernel,
    out_type=(jax.ShapeDtypeStruct((_NPAD, 128), jnp.float32),
              jax.ShapeDtypeStruct((_NPAD, 128), jnp.float32)),
    mesh=plsc.VectorSubcoreMesh(core_axis_name="c", subcore_axis_name="s"),
    scratch_types=[
        pltpu.VMEM((_CHUNK,), jnp.int32),
        pltpu.VMEM((_CHUNK,), jnp.int32),
        pltpu.VMEM((_CHUNK,), jnp.int32),
        pltpu.VMEM((_CHUNK,), jnp.int32),
        pltpu.VMEM((_CHUNK, 128), jnp.float32),
        pltpu.VMEM((_CHUNK, 128), jnp.float32),
        pltpu.VMEM((_ZROWS, 128), jnp.float32),
        pltpu.VMEM_SHARED((_NPAD, 128), jnp.float32),
        pltpu.SemaphoreType.DMA,
        pltpu.SemaphoreType.DMA,
    ],
)(_scatter_edges_body)


def _dense_body(x_ref, w1_ref, b1_ref, w2_ref, b2_ref, wc1_ref, deg_ref, y_ref):
    h = _leaky(jnp.dot(x_ref[...], w1_ref[...],
                       preferred_element_type=jnp.float32) + b1_ref[...])
    h = _leaky(jnp.dot(h, w2_ref[...],
                       preferred_element_type=jnp.float32) + b2_ref[...])
    xw = jnp.dot(h, wc1_ref[...], preferred_element_type=jnp.float32)
    dis = lax.rsqrt(deg_ref[...] + 1.0)
    y_ref[...] = xw * dis


def _mid_body(agg0_ref, agg1_ref, y_ref, deg_ref, bc1_ref, wc2_ref,
              e1_ref, y2_ref):
    dis = lax.rsqrt(deg_ref[...] + 1.0)
    agg = jnp.concatenate([agg0_ref[...], agg1_ref[...]], axis=1)
    e1 = _leaky(dis * (agg + y_ref[...]) + bc1_ref[...])
    e1_ref[...] = e1
    y2_ref[...] = jnp.dot(e1, wc2_ref[...],
                          preferred_element_type=jnp.float32) * dis


def _final_body(agg0_ref, agg1_ref, y2_ref, deg_ref, bc2_ref, e2_ref):
    dis = lax.rsqrt(deg_ref[...] + 1.0)
    agg = agg0_ref[...] + agg1_ref[...]
    e2_ref[...] = _leaky(dis * (agg + y2_ref[...]) + bc2_ref[...])


def _row_spec(w):
    return pl.BlockSpec((_BM, w), lambda i: (i, 0))


def _full_spec(h, w):
    return pl.BlockSpec((h, w), lambda i: (0, 0))


_GRID = _N // _BM

_dense = pl.pallas_call(
    _dense_body,
    grid=(_GRID,),
    in_specs=[_row_spec(128), _full_spec(128, 256), _full_spec(1, 256),
              _full_spec(256, 256), _full_spec(1, 256), _full_spec(256, 256),
              _row_spec(1)],
    out_specs=_row_spec(256),
    out_shape=jax.ShapeDtypeStruct((_N, 256), jnp.float32),
)

_mid = pl.pallas_call(
    _mid_body,
    grid=(_GRID,),
    in_specs=[_row_spec(128), _row_spec(128), _row_spec(256), _row_spec(1),
              _full_spec(1, 256), _full_spec(256, 128)],
    out_specs=(_row_spec(256), _row_spec(128)),
    out_shape=(jax.ShapeDtypeStruct((_N, 256), jnp.float32),
               jax.ShapeDtypeStruct((_N, 128), jnp.float32)),
)

_final = pl.pallas_call(
    _final_body,
    grid=(_GRID,),
    in_specs=[_row_spec(128), _row_spec(128), _row_spec(128), _row_spec(1),
              _full_spec(1, 128)],
    out_specs=_row_spec(128),
    out_shape=jax.ShapeDtypeStruct((_N, 128), jnp.float32),
)


def kernel(x, edge_index, W1, b1, W2, b2, Wc1, bc1, Wc2, bc2, prev1, prev2):
    npad = _EPAD - _E
    src = jnp.concatenate([edge_index[0], jnp.zeros((npad,), jnp.int32)])
    pad_dst = _N + jnp.arange(npad, dtype=jnp.int32) % (_NPAD - _N)
    dst = jnp.concatenate([edge_index[1], pad_dst])
    dst2 = dst.reshape(_NCHUNK, _CHUNK)
    dp = _deg(dst2)
    deg_col = (dp[0, :_N] + dp[1, :_N]).reshape(_N, 1)
    y1 = _dense(x, W1, b1.reshape(1, -1), W2, b2.reshape(1, -1), Wc1, deg_col)
    a0, a1 = _scatter128(y1.reshape(2 * _N, 128), src, dst)
    e1, y2 = _mid(a0[:_N], a1[:_N], y1, deg_col, bc1.reshape(1, -1), Wc2)
    c0, c1 = _scatter_edges(y2, src, dst)
    e2 = _final(c0[:_N], c1[:_N], y2, deg_col, bc2.reshape(1, -1))
    return (e1, e2)


# speedup vs baseline: 1.6539x; 1.6449x over previous
import functools

import jax
import jax.numpy as jnp
from jax import lax
from jax.experimental import pallas as pl
from jax.experimental.pallas import tpu as pltpu
from jax.experimental.pallas import tpu_sc as plsc

_N = 10000
_NPAD = 10112
_E = 320000
_CHUNK = 128
_NS = 16
_CPT = 160
_EPAD = _CHUNK * _NS * _CPT
_NCHUNK = _EPAD // _CHUNK
_RPT = _NPAD // _NS
_ZROWS = 128
_BM = 400
_SCHUNK = _E // _CHUNK
_SCPT = -(-_SCHUNK // _NS)


def _leaky(t):
    return jnp.where(t >= 0, t, 0.01 * t)


_DCPW = _NCHUNK // 32


def _deg_body(dst_hbm, out_hbm, didx_all, d0, d1, d2, d3, d4, d5, d6, d7,
              ones, zbuf, acc, sem):
    dbufs = (d0, d1, d2, d3, d4, d5, d6, d7)
    c = lax.axis_index("c")
    s = lax.axis_index("s")
    wid = c * _NS + s

    def fillz(i, carry):
        zbuf[pl.ds(i * 16, 16)] = jnp.zeros((16,), jnp.float32)
        return carry

    lax.fori_loop(0, 40, fillz, 0)

    def fill1(i, carry):
        ones[pl.ds(i * 16, 16)] = jnp.ones((16,), jnp.float32)
        return carry

    lax.fori_loop(0, _CHUNK // 16, fill1, 0)

    pltpu.sync_copy(zbuf.at[pl.ds(0, _RPT)], acc.at[pl.ds(s * _RPT, _RPT)])
    pltpu.sync_copy(dst_hbm.at[pl.ds(wid * _DCPW, _DCPW)], didx_all)
    plsc.subcore_barrier()

    def wave(j8, carry):
        for b in range(8):
            j = j8 * 8 + b
            for t in range(_CHUNK // 16):
                dbufs[b][pl.ds(t * 16, 16)] = didx_all[j, pl.ds(t * 16, 16)]
            pltpu.async_copy(ones, acc.at[dbufs[b]], sem, add=True)
        for b in range(8):
            pltpu.make_async_copy(ones, acc.at[dbufs[0]], sem).wait()
        return carry

    lax.fori_loop(0, _DCPW // 8, wave, 0)
    plsc.subcore_barrier()

    @pl.when(s == 0)
    def _():
        pltpu.sync_copy(acc, out_hbm.at[c])


_deg = functools.partial(
    pl.kernel,
    out_type=jax.ShapeDtypeStruct((2, _NPAD), jnp.float32),
    mesh=plsc.VectorSubcoreMesh(core_axis_name="c", subcore_axis_name="s"),
    scratch_types=[
        pltpu.VMEM((_DCPW, _CHUNK), jnp.int32),
    ] + [pltpu.VMEM((_CHUNK,), jnp.int32)] * 8 + [
        pltpu.VMEM((_CHUNK,), jnp.float32),
        pltpu.VMEM((640,), jnp.float32),
        pltpu.VMEM_SHARED((_NPAD,), jnp.float32),
        pltpu.SemaphoreType.DMA,
    ],
)(_deg_body)


def _zero_acc(zbuf, acc, s, hw):
    def fillz(i, carry):
        for l in range(hw // 16):
            zbuf[i, pl.ds(l * 16, 16)] = jnp.zeros((16,), jnp.float32)
        return carry

    lax.fori_loop(0, _ZROWS, fillz, 0)
    for k in range(_RPT // _ZROWS):
        pltpu.sync_copy(zbuf, acc.at[pl.ds(s * _RPT + k * _ZROWS, _ZROWS)])
    tail = _RPT - (_RPT // _ZROWS) * _ZROWS
    if tail:
        pltpu.sync_copy(
            zbuf.at[pl.ds(0, tail)],
            acc.at[pl.ds(s * _RPT + (_RPT // _ZROWS) * _ZROWS, tail)])


def _scatter128_body(y_hbm, src_hbm, dst_hbm, out0, out1,
                     si0, si1, gi0, gi1, di0, di1,
                     rb0, rb1, zbuf, acc, gs0, gs1):
    c = lax.axis_index("c")
    s = lax.axis_index("s")

    _zero_acc(zbuf, acc, s, 128)
    plsc.subcore_barrier()

    def chunk(j, carry):
        ch = s + _NS * j

        @pl.when(ch < _SCHUNK)
        def _():
            pltpu.sync_copy(src_hbm.at[pl.ds(ch * _CHUNK, _CHUNK)], si0)
            pltpu.sync_copy(dst_hbm.at[pl.ds(ch * _CHUNK, _CHUNK)], di0)
            for t in range(_CHUNK // 16):
                sv = si0[pl.ds(t * 16, 16)]
                gi0[pl.ds(t * 16, 16)] = sv * 2 + c
            pltpu.async_copy(y_hbm.at[gi0], rb0, gs0).wait()
            pltpu.sync_copy(rb0, acc.at[di0], add=True)

        return carry

    lax.fori_loop(0, _SCPT, chunk, 0)
    plsc.subcore_barrier()

    @pl.when(c == 0)
    def _():
        pltpu.sync_copy(acc.at[pl.ds(s * _RPT, _RPT)],
                        out0.at[pl.ds(s * _RPT, _RPT)])

    @pl.when(c == 1)
    def _():
        pltpu.sync_copy(acc.at[pl.ds(s * _RPT, _RPT)],
                        out1.at[pl.ds(s * _RPT, _RPT)])


_scatter128 = functools.partial(
    pl.kernel,
    out_type=(jax.ShapeDtypeStruct((_NPAD, 128), jnp.float32),
              jax.ShapeDtypeStruct((_NPAD, 128), jnp.float32)),
    mesh=plsc.VectorSubcoreMesh(core_axis_name="c", subcore_axis_name="s"),
    scratch_types=[
        pltpu.VMEM((_CHUNK,), jnp.int32),
        pltpu.VMEM((_CHUNK,), jnp.int32),
        pltpu.VMEM((_CHUNK,), jnp.int32),
        pltpu.VMEM((_CHUNK,), jnp.int32),
        pltpu.VMEM((_CHUNK,), jnp.int32),
        pltpu.VMEM((_CHUNK,), jnp.int32),
        pltpu.VMEM((_CHUNK, 128), jnp.float32),
        pltpu.VMEM((_CHUNK, 128), jnp.float32),
        pltpu.VMEM((_ZROWS, 128), jnp.float32),
        pltpu.VMEM_SHARED((_NPAD, 128), jnp.float32),
        pltpu.SemaphoreType.DMA,
        pltpu.SemaphoreType.DMA,
    ],
)(_scatter128_body)


_ECPT = _NCHUNK // 32


def _scatter_edges_body(y_hbm, src_hbm, dst_hbm, out0, out1,
                        si0, si1, di0, di1,
                        rb0, rb1, zbuf, acc, gs0, gs1):
    c = lax.axis_index("c")
    s = lax.axis_index("s")

    _zero_acc(zbuf, acc, s, 128)
    plsc.subcore_barrier()

    def chunk(j, carry):
        half = s + _NS * j

        @pl.when(half < _SCHUNK // 2)
        def _():
            ch = 2 * half + c
            pltpu.sync_copy(src_hbm.at[pl.ds(ch * _CHUNK, _CHUNK)], si0)
            pltpu.sync_copy(dst_hbm.at[pl.ds(ch * _CHUNK, _CHUNK)], di0)
            pltpu.async_copy(y_hbm.at[si0], rb0, gs0).wait()
            pltpu.sync_copy(rb0, acc.at[di0], add=True)

        return carry

    lax.fori_loop(0, -(-(_SCHUNK // 2) // _NS), chunk, 0)
    plsc.subcore_barrier()

    @pl.when(c == 0)
    def _():
        pltpu.sync_copy(acc.at[pl.ds(s * _RPT, _RPT)],
                        out0.at[pl.ds(s * _RPT, _RPT)])

    @pl.when(c == 1)
    def _():
        pltpu.sync_copy(acc.at[pl.ds(s * _RPT, _RPT)],
                        out1.at[pl.ds(s * _RPT, _RPT)])


_scatter_edges = functools.partial(
    pl.kernel,
    out_type=(jax.ShapeDtypeStruct((_NPAD, 128), jnp.float32),
              jax.ShapeDtypeStruct((_NPAD, 128), jnp.float32)),
    mesh=plsc.VectorSubcoreMesh(core_axis_name="c", subcore_axis_name="s"),
    scratch_types=[
        pltpu.VMEM((_CHUNK,), jnp.int32),
        pltpu.VMEM((_CHUNK,), jnp.int32),
        pltpu.VMEM((_CHUNK,), jnp.int32),
        pltpu.VMEM((_CHUNK,), jnp.int32),
        pltpu.VMEM((_CHUNK, 128), jnp.float32),
        pltpu.VMEM((_CHUNK, 128), jnp.float32),
        pltpu.VMEM((_ZROWS, 128), jnp.float32),
        pltpu.VMEM_SHARED((_NPAD, 128), jnp.float32),
        pltpu.SemaphoreType.DMA,
        pltpu.SemaphoreType.DMA,
    ],
)(_scatter_edges_body)


def _dense_body(x_ref, w1_ref, b1_ref, w2_ref, b2_ref, wc1_ref, deg_ref, y_ref):
    h = _leaky(jnp.dot(x_ref[...], w1_ref[...],
                       preferred_element_type=jnp.float32) + b1_ref[...])
    h = _leaky(jnp.dot(h, w2_ref[...],
                       preferred_element_type=jnp.float32) + b2_ref[...])
    xw = jnp.dot(h, wc1_ref[...], preferred_element_type=jnp.float32)
    dis = lax.rsqrt(deg_ref[...] + 1.0)
    y_ref[...] = xw * dis


def _mid_body(agg0_ref, agg1_ref, y_ref, deg_ref, bc1_ref, wc2_ref,
              e1_ref, y2_ref):
    dis = lax.rsqrt(deg_ref[...] + 1.0)
    agg = jnp.concatenate([agg0_ref[...], agg1_ref[...]], axis=1)
    e1 = _leaky(dis * (agg + y_ref[...]) + bc1_ref[...])
    e1_ref[...] = e1
    y2_ref[...] = jnp.dot(e1, wc2_ref[...],
                          preferred_element_type=jnp.float32) * dis


def _final_body(agg0_ref, agg1_ref, y2_ref, deg_ref, bc2_ref, e2_ref):
    dis = lax.rsqrt(deg_ref[...] + 1.0)
    agg = agg0_ref[...] + agg1_ref[...]
    e2_ref[...] = _leaky(dis * (agg + y2_ref[...]) + bc2_ref[...])


def _row_spec(w):
    return pl.BlockSpec((_BM, w), lambda i: (i, 0))


def _full_spec(h, w):
    return pl.BlockSpec((h, w), lambda i: (0, 0))


_GRID = _N // _BM

_dense = pl.pallas_call(
    _dense_body,
    grid=(_GRID,),
    in_specs=[_row_spec(128), _full_spec(128, 256), _full_spec(1, 256),
              _full_spec(256, 256), _full_spec(1, 256), _full_spec(256, 256),
              _row_spec(1)],
    out_specs=_row_spec(256),
    out_shape=jax.ShapeDtypeStruct((_N, 256), jnp.float32),
)

_mid = pl.pallas_call(
    _mid_body,
    grid=(_GRID,),
    in_specs=[_row_spec(128), _row_spec(128), _row_spec(256), _row_spec(1),
              _full_spec(1, 256), _full_spec(256, 128)],
    out_specs=(_row_spec(256), _row_spec(128)),
    out_shape=(jax.ShapeDtypeStruct((_N, 256), jnp.float32),
               jax.ShapeDtypeStruct((_N, 128), jnp.float32)),
)

_final = pl.pallas_call(
    _final_body,
    grid=(_GRID,),
    in_specs=[_row_spec(128), _row_spec(128), _row_spec(128), _row_spec(1),
              _full_spec(1, 128)],
    out_specs=_row_spec(128),
    out_shape=jax.ShapeDtypeStruct((_N, 128), jnp.float32),
)


def kernel(x, edge_index, W1, b1, W2, b2, Wc1, bc1, Wc2, bc2, prev1, prev2):
    npad = _EPAD - _E
    src = edge_index[0]
    dst = edge_index[1]
    pad_dst = _N + jnp.arange(npad, dtype=jnp.int32) % (_NPAD - _N)
    dst2 = jnp.concatenate([dst, pad_dst]).reshape(_NCHUNK, _CHUNK)
    dp = _deg(dst2)
    deg_col = (dp[0, :_N] + dp[1, :_N]).reshape(_N, 1)
    y1 = _dense(x, W1, b1.reshape(1, -1), W2, b2.reshape(1, -1), Wc1, deg_col)
    a0, a1 = _scatter128(y1.reshape(2 * _N, 128), src, dst)
    e1, y2 = _mid(a0[:_N], a1[:_N], y1, deg_col, bc1.reshape(1, -1), Wc2)
    c0, c1 = _scatter_edges(y2, src, dst)
    e2 = _final(c0[:_N], c1[:_N], y2, deg_col, bc2.reshape(1, -1))
    return (e1, e2)


# speedup vs baseline: 2.5180x; 1.5225x over previous
import functools

import jax
import jax.numpy as jnp
from jax import lax
from jax.experimental import pallas as pl
from jax.experimental.pallas import tpu as pltpu
from jax.experimental.pallas import tpu_sc as plsc

_N = 10000
_NPAD = 10112
_E = 320000
_CHUNK = 128
_NS = 16
_CPT = 160
_EPAD = _CHUNK * _NS * _CPT
_NCHUNK = _EPAD // _CHUNK
_RPT = _NPAD // _NS
_ZROWS = 128
_BM = 400
_SCHUNK = _E // _CHUNK
_SCPT = -(-_SCHUNK // _NS)


def _leaky(t):
    return jnp.where(t >= 0, t, 0.01 * t)


_DCPW = _NCHUNK // 32


def _deg_body(dst_hbm, out_hbm, didx_all, d0, d1, d2, d3, d4, d5, d6, d7,
              ones, zbuf, acc, sem):
    dbufs = (d0, d1, d2, d3, d4, d5, d6, d7)
    c = lax.axis_index("c")
    s = lax.axis_index("s")
    wid = c * _NS + s

    def fillz(i, carry):
        zbuf[pl.ds(i * 16, 16)] = jnp.zeros((16,), jnp.float32)
        return carry

    lax.fori_loop(0, 40, fillz, 0)

    def fill1(i, carry):
        ones[pl.ds(i * 16, 16)] = jnp.ones((16,), jnp.float32)
        return carry

    lax.fori_loop(0, _CHUNK // 16, fill1, 0)

    pltpu.sync_copy(zbuf.at[pl.ds(0, _RPT)], acc.at[pl.ds(s * _RPT, _RPT)])
    pltpu.sync_copy(dst_hbm.at[pl.ds(wid * _DCPW, _DCPW)], didx_all)
    plsc.subcore_barrier()

    def wave(j8, carry):
        for b in range(8):
            j = j8 * 8 + b
            for t in range(_CHUNK // 16):
                dbufs[b][pl.ds(t * 16, 16)] = didx_all[j, pl.ds(t * 16, 16)]
            pltpu.async_copy(ones, acc.at[dbufs[b]], sem, add=True)
        for b in range(8):
            pltpu.make_async_copy(ones, acc.at[dbufs[0]], sem).wait()
        return carry

    lax.fori_loop(0, _DCPW // 8, wave, 0)
    plsc.subcore_barrier()

    @pl.when(s == 0)
    def _():
        pltpu.sync_copy(acc, out_hbm.at[c])


_deg = functools.partial(
    pl.kernel,
    out_type=jax.ShapeDtypeStruct((2, _NPAD), jnp.float32),
    mesh=plsc.VectorSubcoreMesh(core_axis_name="c", subcore_axis_name="s"),
    scratch_types=[
        pltpu.VMEM((_DCPW, _CHUNK), jnp.int32),
    ] + [pltpu.VMEM((_CHUNK,), jnp.int32)] * 8 + [
        pltpu.VMEM((_CHUNK,), jnp.float32),
        pltpu.VMEM((640,), jnp.float32),
        pltpu.VMEM_SHARED((_NPAD,), jnp.float32),
        pltpu.SemaphoreType.DMA,
    ],
)(_deg_body)


def _zero_acc(zbuf, acc, s, hw):
    def fillz(i, carry):
        for l in range(hw // 16):
            zbuf[i, pl.ds(l * 16, 16)] = jnp.zeros((16,), jnp.float32)
        return carry

    lax.fori_loop(0, _ZROWS, fillz, 0)
    for k in range(_RPT // _ZROWS):
        pltpu.sync_copy(zbuf, acc.at[pl.ds(s * _RPT + k * _ZROWS, _ZROWS)])
    tail = _RPT - (_RPT // _ZROWS) * _ZROWS
    if tail:
        pltpu.sync_copy(
            zbuf.at[pl.ds(0, tail)],
            acc.at[pl.ds(s * _RPT + (_RPT // _ZROWS) * _ZROWS, tail)])


def _scatter128_body(y_hbm, src_hbm, dst_hbm, out0, out1,
                     si0, si1, gi0, gi1, di0, di1,
                     rb0, rb1, zbuf, acc, gs0, gs1):
    c = lax.axis_index("c")
    s = lax.axis_index("s")

    _zero_acc(zbuf, acc, s, 128)
    plsc.subcore_barrier()

    sis, gis, dis_ = (si0, si1), (gi0, gi1), (di0, di1)
    rbs, gss = (rb0, rb1), (gs0, gs1)

    def prep(j, b):
        ch = s + _NS * j
        pltpu.sync_copy(src_hbm.at[pl.ds(ch * _CHUNK, _CHUNK)], sis[b])
        pltpu.sync_copy(dst_hbm.at[pl.ds(ch * _CHUNK, _CHUNK)], dis_[b])
        for t in range(_CHUNK // 16):
            sv = sis[b][pl.ds(t * 16, 16)]
            gis[b][pl.ds(t * 16, 16)] = sv * 2 + c
        pltpu.async_copy(y_hbm.at[gis[b]], rbs[b], gss[b])

    def finish(b):
        pltpu.make_async_copy(y_hbm.at[gis[b]], rbs[b], gss[b]).wait()
        pltpu.sync_copy(rbs[b], acc.at[dis_[b]], add=True)

    def valid(j):
        return s + _NS * j < _SCHUNK

    prep(0, 0)

    def pair(j2, carry):
        j = 2 * j2

        @pl.when(valid(j + 1))
        def _():
            prep(j + 1, 1)

        @pl.when(valid(j))
        def _():
            finish(0)

        @pl.when(valid(j + 2))
        def _():
            prep(j + 2, 0)

        @pl.when(valid(j + 1))
        def _():
            finish(1)

        return carry

    lax.fori_loop(0, (_SCPT + 1) // 2, pair, 0)
    plsc.subcore_barrier()

    @pl.when(c == 0)
    def _():
        pltpu.sync_copy(acc.at[pl.ds(s * _RPT, _RPT)],
                        out0.at[pl.ds(s * _RPT, _RPT)])

    @pl.when(c == 1)
    def _():
        pltpu.sync_copy(acc.at[pl.ds(s * _RPT, _RPT)],
                        out1.at[pl.ds(s * _RPT, _RPT)])


_scatter128 = functools.partial(
    pl.kernel,
    out_type=(jax.ShapeDtypeStruct((_NPAD, 128), jnp.float32),
              jax.ShapeDtypeStruct((_NPAD, 128), jnp.float32)),
    mesh=plsc.VectorSubcoreMesh(core_axis_name="c", subcore_axis_name="s"),
    scratch_types=[
        pltpu.VMEM((_CHUNK,), jnp.int32),
        pltpu.VMEM((_CHUNK,), jnp.int32),
        pltpu.VMEM((_CHUNK,), jnp.int32),
        pltpu.VMEM((_CHUNK,), jnp.int32),
        pltpu.VMEM((_CHUNK,), jnp.int32),
        pltpu.VMEM((_CHUNK,), jnp.int32),
        pltpu.VMEM((_CHUNK, 128), jnp.float32),
        pltpu.VMEM((_CHUNK, 128), jnp.float32),
        pltpu.VMEM((_ZROWS, 128), jnp.float32),
        pltpu.VMEM_SHARED((_NPAD, 128), jnp.float32),
        pltpu.SemaphoreType.DMA,
        pltpu.SemaphoreType.DMA,
    ],
)(_scatter128_body)


_ECPT = _NCHUNK // 32


def _scatter_edges_body(y_hbm, src_hbm, dst_hbm, out0, out1,
                        si0, si1, di0, di1,
                        rb0, rb1, zbuf, acc, gs0, gs1):
    c = lax.axis_index("c")
    s = lax.axis_index("s")

    _zero_acc(zbuf, acc, s, 128)
    plsc.subcore_barrier()

    sis, dis_ = (si0, si1), (di0, di1)
    rbs, gss = (rb0, rb1), (gs0, gs1)

    def prep(j, b):
        ch = 2 * (s + _NS * j) + c
        pltpu.sync_copy(src_hbm.at[pl.ds(ch * _CHUNK, _CHUNK)], sis[b])
        pltpu.sync_copy(dst_hbm.at[pl.ds(ch * _CHUNK, _CHUNK)], dis_[b])
        pltpu.async_copy(y_hbm.at[sis[b]], rbs[b], gss[b])

    def finish(b):
        pltpu.make_async_copy(y_hbm.at[sis[b]], rbs[b], gss[b]).wait()
        pltpu.sync_copy(rbs[b], acc.at[dis_[b]], add=True)

    def valid(j):
        return s + _NS * j < _SCHUNK // 2

    prep(0, 0)

    def pair(j2, carry):
        j = 2 * j2

        @pl.when(valid(j + 1))
        def _():
            prep(j + 1, 1)

        @pl.when(valid(j))
        def _():
            finish(0)

        @pl.when(valid(j + 2))
        def _():
            prep(j + 2, 0)

        @pl.when(valid(j + 1))
        def _():
            finish(1)

        return carry

    npairs = (-(-(_SCHUNK // 2) // _NS) + 1) // 2
    lax.fori_loop(0, npairs, pair, 0)
    plsc.subcore_barrier()

    @pl.when(c == 0)
    def _():
        pltpu.sync_copy(acc.at[pl.ds(s * _RPT, _RPT)],
                        out0.at[pl.ds(s * _RPT, _RPT)])

    @pl.when(c == 1)
    def _():
        pltpu.sync_copy(acc.at[pl.ds(s * _RPT, _RPT)],
                        out1.at[pl.ds(s * _RPT, _RPT)])


_scatter_edges = functools.partial(
    pl.kernel,
    out_type=(jax.ShapeDtypeStruct((_NPAD, 128), jnp.float32),
              jax.ShapeDtypeStruct((_NPAD, 128), jnp.float32)),
    mesh=plsc.VectorSubcoreMesh(core_axis_name="c", subcore_axis_name="s"),
    scratch_types=[
        pltpu.VMEM((_CHUNK,), jnp.int32),
        pltpu.VMEM((_CHUNK,), jnp.int32),
        pltpu.VMEM((_CHUNK,), jnp.int32),
        pltpu.VMEM((_CHUNK,), jnp.int32),
        pltpu.VMEM((_CHUNK, 128), jnp.float32),
        pltpu.VMEM((_CHUNK, 128), jnp.float32),
        pltpu.VMEM((_ZROWS, 128), jnp.float32),
        pltpu.VMEM_SHARED((_NPAD, 128), jnp.float32),
        pltpu.SemaphoreType.DMA,
        pltpu.SemaphoreType.DMA,
    ],
)(_scatter_edges_body)


def _dense_body(x_ref, w1_ref, b1_ref, w2_ref, b2_ref, wc1_ref, deg_ref, y_ref):
    h = _leaky(jnp.dot(x_ref[...], w1_ref[...],
                       preferred_element_type=jnp.float32) + b1_ref[...])
    h = _leaky(jnp.dot(h, w2_ref[...],
                       preferred_element_type=jnp.float32) + b2_ref[...])
    xw = jnp.dot(h, wc1_ref[...], preferred_element_type=jnp.float32)
    dis = lax.rsqrt(deg_ref[...] + 1.0)
    y_ref[...] = xw * dis


def _mid_body(agg0_ref, agg1_ref, y_ref, deg_ref, bc1_ref, wc2_ref,
              e1_ref, y2_ref):
    dis = lax.rsqrt(deg_ref[...] + 1.0)
    agg = jnp.concatenate([agg0_ref[...], agg1_ref[...]], axis=1)
    e1 = _leaky(dis * (agg + y_ref[...]) + bc1_ref[...])
    e1_ref[...] = e1
    y2_ref[...] = jnp.dot(e1, wc2_ref[...],
                          preferred_element_type=jnp.float32) * dis


def _final_body(agg0_ref, agg1_ref, y2_ref, deg_ref, bc2_ref, e2_ref):
    dis = lax.rsqrt(deg_ref[...] + 1.0)
    agg = agg0_ref[...] + agg1_ref[...]
    e2_ref[...] = _leaky(dis * (agg + y2_ref[...]) + bc2_ref[...])


def _row_spec(w):
    return pl.BlockSpec((_BM, w), lambda i: (i, 0))


def _full_spec(h, w):
    return pl.BlockSpec((h, w), lambda i: (0, 0))


_GRID = _N // _BM

_dense = pl.pallas_call(
    _dense_body,
    grid=(_GRID,),
    in_specs=[_row_spec(128), _full_spec(128, 256), _full_spec(1, 256),
              _full_spec(256, 256), _full_spec(1, 256), _full_spec(256, 256),
              _row_spec(1)],
    out_specs=_row_spec(256),
    out_shape=jax.ShapeDtypeStruct((_N, 256), jnp.float32),
)

_mid = pl.pallas_call(
    _mid_body,
    grid=(_GRID,),
    in_specs=[_row_spec(128), _row_spec(128), _row_spec(256), _row_spec(1),
              _full_spec(1, 256), _full_spec(256, 128)],
    out_specs=(_row_spec(256), _row_spec(128)),
    out_shape=(jax.ShapeDtypeStruct((_N, 256), jnp.float32),
               jax.ShapeDtypeStruct((_N, 128), jnp.float32)),
)

_final = pl.pallas_call(
    _final_body,
    grid=(_GRID,),
    in_specs=[_row_spec(128), _row_spec(128), _row_spec(128), _row_spec(1),
              _full_spec(1, 128)],
    out_specs=_row_spec(128),
    out_shape=jax.ShapeDtypeStruct((_N, 128), jnp.float32),
)


def kernel(x, edge_index, W1, b1, W2, b2, Wc1, bc1, Wc2, bc2, prev1, prev2):
    npad = _EPAD - _E
    src = edge_index[0]
    dst = edge_index[1]
    pad_dst = _N + jnp.arange(npad, dtype=jnp.int32) % (_NPAD - _N)
    dst2 = jnp.concatenate([dst, pad_dst]).reshape(_NCHUNK, _CHUNK)
    dp = _deg(dst2)
    deg_col = (dp[0, :_N] + dp[1, :_N]).reshape(_N, 1)
    y1 = _dense(x, W1, b1.reshape(1, -1), W2, b2.reshape(1, -1), Wc1, deg_col)
    a0, a1 = _scatter128(y1.reshape(2 * _N, 128), src, dst)
    e1, y2 = _mid(a0[:_N], a1[:_N], y1, deg_col, bc1.reshape(1, -1), Wc2)
    c0, c1 = _scatter_edges(y2, src, dst)
    e2 = _final(c0[:_N], c1[:_N], y2, deg_col, bc2.reshape(1, -1))
    return (e1, e2)
